# baseline (speedup 1.0000x reference)
import functools

import jax
import jax.numpy as jnp
from jax import lax
from jax.experimental import pallas as pl
from jax.experimental.pallas import tpu as pltpu
from jax.experimental.pallas import tpu_sc as plsc

VOCAB_ROWS = 8021
DIM = 312
PIECE = 104
PIECE_PAD = 128
NPIECE = 3
B = 4096
T = 200
NUM_IDX = B * T

NUM_CORES = 2
NUM_SUBCORES = 16
NUM_WORKERS = NUM_CORES * NUM_SUBCORES

BLK = 128
BB = B // BLK
NBLOCKS = T * BB
BLOCKS_PER_WORKER = NBLOCKS // NUM_WORKERS


def _transpose_piece(rows_p, buft_p):
    jj = jax.lax.iota(jnp.int32, 16)

    perms = [(jj + d) & 15 for d in range(16)]

    def tile_row(c0i, carry):
        c0 = c0i * 16
        bs = [c0 + p for p in perms]

        def lgroup(l0i, carry2):
            a = l0i * 16 + jj
            for d in range(16):
                vec = plsc.load_gather(rows_p, [a, bs[d]])
                plsc.store_scatter(buft_p, [bs[d], a], vec)
            return carry2

        lax.fori_loop(0, BLK // 16, lgroup, 0)
        return carry

    lax.fori_loop(0, PIECE // 16, tile_row, 0)

    def diag8(d, carry):
        b = (PIECE - 8) + ((jj + d) & 7)
        for l0 in range(0, BLK, 16):
            a = l0 + jj
            vec = plsc.load_gather(rows_p, [a, b])
            plsc.store_scatter(buft_p, [b, a], vec)
        return carry

    lax.fori_loop(0, 8, diag8, 0)


def _gather_body(table_hbm, idx_hbm, out1_hbm, out2_hbm, idx_b, idxp, rows,
                 buft, isems, gsems, wsems):
    wid = lax.axis_index("s") * NUM_CORES + lax.axis_index("c")
    blk_base = wid * BLOCKS_PER_WORKER

    def start_idx(s, slot):
        pltpu.async_copy(idx_hbm.at[pl.ds((blk_base + s) * BLK, BLK)],
                         idx_b.at[slot], isems.at[slot])

    def wait_idx(slot):
        pltpu.make_async_copy(idx_hbm.at[pl.ds(0, BLK)], idx_b.at[slot],
                              isems.at[slot]).wait()

    def compute_idxp(slot):
        for p in range(NPIECE):
            for k in range(0, BLK, 16):
                v = idx_b[slot, pl.ds(k, 16)]
                idxp[p, pl.ds(k, 16)] = v + p * VOCAB_ROWS

    def start_gather(p):
        pltpu.async_copy(table_hbm.at[idxp.at[p]], rows.at[p], gsems.at[p])

    def wait_gather(p):
        pltpu.make_async_copy(table_hbm.at[idxp.at[0]], rows.at[p],
                              gsems.at[p]).wait()

    def start_writes(s, p):
        beta = blk_base + s
        t = beta // BB
        bb = beta % BB
        for out in (out1_hbm, out2_hbm):
            pltpu.async_copy(
                buft.at[p],
                out.at[t, pl.ds(p * PIECE, PIECE), pl.ds(bb * BLK, BLK)],
                wsems.at[p])

    def wait_writes(p):
        for out in (out1_hbm, out2_hbm):
            pltpu.make_async_copy(
                buft.at[p],
                out.at[0, pl.ds(p * PIECE, PIECE), pl.ds(0, BLK)],
                wsems.at[p]).wait()

    start_idx(0, 0)
    wait_idx(0)
    compute_idxp(0)
    for p in range(NPIECE):
        start_gather(p)
    start_idx(1, 1)

    def step(s, carry):
        for p in range(NPIECE):
            wait_gather(p)

            @pl.when(s > 0)
            def _():
                wait_writes(p)
            _transpose_piece(rows.at[p], buft.at[p])
            start_writes(s, p)

        @pl.when(s + 1 < BLOCKS_PER_WORKER)
        def _():
            slot = lax.rem(s + 1, 2)
            wait_idx(slot)
            compute_idxp(slot)
            for p in range(NPIECE):
                start_gather(p)

        @pl.when(s + 2 < BLOCKS_PER_WORKER)
        def _():
            slot2 = lax.rem(s, 2)
            start_idx(s + 2, slot2)
        return carry

    lax.fori_loop(0, BLOCKS_PER_WORKER, step, 0)

    for p in range(NPIECE):
        wait_writes(p)


@jax.jit
def _embedding_gather(table, idx):
    mesh = plsc.VectorSubcoreMesh(core_axis_name="c", subcore_axis_name="s")
    out_t = jax.ShapeDtypeStruct((T, DIM, B), jnp.float32)
    run = functools.partial(
        pl.kernel,
        out_type=(out_t, out_t),
        mesh=mesh,
        scratch_types=[
            pltpu.VMEM((2, BLK), jnp.int32),
            pltpu.VMEM((NPIECE, BLK), jnp.int32),
            pltpu.VMEM((NPIECE, BLK, PIECE_PAD), jnp.float32),
            pltpu.VMEM((NPIECE, PIECE, BLK), jnp.float32),
            pltpu.SemaphoreType.DMA((2,)),
            pltpu.SemaphoreType.DMA((NPIECE,)),
            pltpu.SemaphoreType.DMA((NPIECE,)),
        ],
        compiler_params=pltpu.CompilerParams(use_tc_tiling_on_sc=True,
                                             needs_layout_passes=False),
    )(_gather_body)
    return run(table, idx)


def kernel(x, mask, emb):
    idx = x.T.reshape(-1).astype(jnp.int32)
    table = jnp.pad(
        jnp.transpose(emb.reshape(VOCAB_ROWS, NPIECE, PIECE), (1, 0, 2)),
        ((0, 0), (0, 0), (0, PIECE_PAD - PIECE))).reshape(
            NPIECE * VOCAB_ROWS, PIECE_PAD)
    f1, f2 = _embedding_gather(table, idx)
    y1 = jnp.transpose(f1, (2, 0, 1))
    y2 = jnp.transpose(f2, (2, 0, 1))
    return (y1, y2)


# speedup vs baseline: 1.1075x; 1.1075x over previous
import functools

import jax
import jax.numpy as jnp
from jax import lax
from jax.experimental import pallas as pl
from jax.experimental.pallas import tpu as pltpu
from jax.experimental.pallas import tpu_sc as plsc

VOCAB_ROWS = 8021
DIM = 312
DIM_PAD = 384
NUM_IDX = 4096 * 200

NUM_CORES = 2
NUM_SUBCORES = 16
NUM_WORKERS = NUM_CORES * NUM_SUBCORES

CHUNK = 64
NBUF = 4
ROWS_PER_WORKER = NUM_IDX // NUM_WORKERS
CHUNKS_PER_WORKER = ROWS_PER_WORKER // CHUNK
LOOKAHEAD = 2


def _gather_body(table_hbm, idx_hbm, out_hbm, idx_bufs, rows, isems, gsems,
                 wsems):
    wid = lax.axis_index("s") * NUM_CORES + lax.axis_index("c")
    row_base = wid * ROWS_PER_WORKER

    def start_idx(c, b):
        pltpu.async_copy(idx_hbm.at[pl.ds(row_base + c * CHUNK, CHUNK)],
                         idx_bufs.at[b], isems.at[b])

    def wait_idx(b):
        pltpu.make_async_copy(idx_hbm.at[pl.ds(row_base, CHUNK)],
                              idx_bufs.at[b], isems.at[b]).wait()

    def start_gather(b):
        pltpu.async_copy(table_hbm.at[idx_bufs.at[b]], rows.at[b], gsems.at[b])

    def wait_gather(b):
        pltpu.make_async_copy(table_hbm.at[idx_bufs.at[0]], rows.at[b],
                              gsems.at[b]).wait()

    def start_write(c, b):
        pltpu.async_copy(rows.at[b],
                         out_hbm.at[pl.ds(row_base + c * CHUNK, CHUNK)],
                         wsems.at[b])

    def wait_write(b):
        pltpu.make_async_copy(rows.at[b],
                              out_hbm.at[pl.ds(row_base, CHUNK)],
                              wsems.at[b]).wait()

    for c in range(LOOKAHEAD):
        start_idx(c, c % NBUF)
    for c in range(LOOKAHEAD):
        wait_idx(c % NBUF)
        start_gather(c % NBUF)

    def step(s, carry):
        for i in range(NBUF):
            c = s * NBUF + i
            b = c % NBUF
            cf = c + LOOKAHEAD
            bf = cf % NBUF

            @pl.when(cf < CHUNKS_PER_WORKER)
            def _():
                @pl.when(cf >= NBUF)
                def _():
                    wait_write(bf)
                start_idx(cf, bf)
                wait_idx(bf)
                start_gather(bf)

            wait_gather(b)
            start_write(c, b)
        return carry

    lax.fori_loop(0, CHUNKS_PER_WORKER // NBUF, step, 0)

    for b in range(NBUF):
        wait_write(b)


@jax.jit
def _embedding_gather(table, idx):
    mesh = plsc.VectorSubcoreMesh(core_axis_name="c", subcore_axis_name="s")
    run = functools.partial(
        pl.kernel,
        out_type=jax.ShapeDtypeStruct((NUM_IDX, DIM_PAD), jnp.float32),
        mesh=mesh,
        scratch_types=[
            pltpu.VMEM((NBUF, CHUNK), jnp.int32),
            pltpu.VMEM((NBUF, CHUNK, DIM_PAD), jnp.float32),
            pltpu.SemaphoreType.DMA((NBUF,)),
            pltpu.SemaphoreType.DMA((NBUF,)),
            pltpu.SemaphoreType.DMA((NBUF,)),
        ],
    )(_gather_body)
    return run(table, idx)


def kernel(x, mask, emb):
    idx = x.reshape(-1).astype(jnp.int32)
    table = jnp.pad(emb, ((0, 0), (0, DIM_PAD - DIM)))
    flat = _embedding_gather(table, idx)
    y = flat[:, :DIM].reshape(x.shape[0], x.shape[1], DIM)
    return (y, y)
